# baseline (speedup 1.0000x reference)
import functools

import jax
import jax.numpy as jnp
from jax.experimental import pallas as pl
from jax.experimental.pallas import tpu as pltpu

_EPS = 1e-06
_CAPACITY_FACTOR = 1.0


def _route_kernel(x_ref, w_ref, b_ref, masked_ref, partial_ref, *,
                  block_tokens):
    i = pl.program_id(0)

    logits = jax.lax.dot_general(
        x_ref[:], w_ref[:],
        dimension_numbers=(((1,), (1,)), ((), ())),
        preferred_element_type=jnp.float32,
    ) + b_ref[:]

    m = jnp.max(logits, axis=-1, keepdims=True)
    idx = jnp.argmax(logits, axis=-1)[:, None]

    s = jnp.sum(jnp.exp(logits - m), axis=-1, keepdims=True)
    lanes = jax.lax.broadcasted_iota(jnp.int32, logits.shape, 1)
    masked = jnp.where(lanes == idx, 1.0 / s, 0.0)

    masked_ref[:] = masked
    partial_ref[0, 0, :] = jnp.sum(masked, axis=0)


def _norm_kernel(masked_ref, partial_ref, out_ref, *, capacity):
    denom = jnp.sum(partial_ref[:, 0, :], axis=0, keepdims=True) + _EPS
    out_ref[:] = masked_ref[:] / denom * capacity


def kernel(x, W, b):
    tokens, dim = x.shape
    num_experts = W.shape[0]
    capacity = int(_CAPACITY_FACTOR * tokens)

    block_tokens = 1024
    num_blocks = tokens // block_tokens

    masked, partials = pl.pallas_call(
        functools.partial(_route_kernel, block_tokens=block_tokens),
        grid=(num_blocks,),
        in_specs=[
            pl.BlockSpec((block_tokens, dim), lambda i: (i, 0)),
            pl.BlockSpec((num_experts, dim), lambda i: (0, 0)),
            pl.BlockSpec((1, num_experts), lambda i: (0, 0)),
        ],
        out_specs=[
            pl.BlockSpec((block_tokens, num_experts), lambda i: (i, 0)),
            pl.BlockSpec((1, 1, num_experts), lambda i: (i, 0, 0)),
        ],
        out_shape=[
            jax.ShapeDtypeStruct((tokens, num_experts), jnp.float32),
            jax.ShapeDtypeStruct((num_blocks, 1, num_experts), jnp.float32),
        ],
    )(x, W, b.reshape(1, num_experts))

    return pl.pallas_call(
        functools.partial(_norm_kernel, capacity=float(capacity)),
        in_specs=[
            pl.BlockSpec((tokens, num_experts), lambda: (0, 0)),
            pl.BlockSpec((num_blocks, 1, num_experts), lambda: (0, 0, 0)),
        ],
        out_specs=pl.BlockSpec((tokens, num_experts), lambda: (0, 0)),
        out_shape=jax.ShapeDtypeStruct((tokens, num_experts), jnp.float32),
    )(masked, partials)


# speedup vs baseline: 1.0975x; 1.0975x over previous
import functools

import jax
import jax.numpy as jnp
from jax.experimental import pallas as pl
from jax.experimental.pallas import tpu as pltpu

_EPS = 1e-06
_CAPACITY_FACTOR = 1.0


def _gate_kernel(x_ref, w_ref, b_ref, out_ref, denom_ref, *, block_tokens,
                 num_blocks, capacity):
    i = pl.program_id(0)

    logits = jax.lax.dot_general(
        w_ref[:], x_ref[:],
        dimension_numbers=(((1,), (1,)), ((), ())),
        preferred_element_type=jnp.float32,
    ) + b_ref[:]

    m = jnp.max(logits, axis=0, keepdims=True)
    idx = jnp.argmax(logits, axis=0)[None, :]

    s = jnp.sum(jnp.exp(logits - m), axis=0, keepdims=True)
    subl = jax.lax.broadcasted_iota(jnp.int32, logits.shape, 0)
    masked_t = jnp.where(subl == idx, 1.0 / s, 0.0)

    out_ref[pl.ds(i * block_tokens, block_tokens), :] = masked_t.T

    @pl.when(i == 0)
    def _init():
        denom_ref[:] = jnp.zeros_like(denom_ref)

    denom_ref[:] += jnp.sum(masked_t, axis=1, keepdims=True)

    @pl.when(i == num_blocks - 1)
    def _finalize():
        out_ref[:] = out_ref[:] / (denom_ref[:].T + _EPS) * capacity


def kernel(x, W, b):
    tokens, dim = x.shape
    num_experts = W.shape[0]
    capacity = int(_CAPACITY_FACTOR * tokens)

    block_tokens = 1024
    num_blocks = tokens // block_tokens

    body = functools.partial(
        _gate_kernel,
        block_tokens=block_tokens,
        num_blocks=num_blocks,
        capacity=float(capacity),
    )

    return pl.pallas_call(
        body,
        grid=(num_blocks,),
        in_specs=[
            pl.BlockSpec((block_tokens, dim), lambda i: (i, 0)),
            pl.BlockSpec((num_experts, dim), lambda i: (0, 0)),
            pl.BlockSpec((num_experts, 1), lambda i: (0, 0)),
        ],
        out_specs=pl.BlockSpec((tokens, num_experts), lambda i: (0, 0)),
        out_shape=jax.ShapeDtypeStruct((tokens, num_experts), jnp.float32),
        scratch_shapes=[pltpu.VMEM((num_experts, 1), jnp.float32)],
    )(x, W, b.reshape(num_experts, 1))
